# baseline (speedup 1.0000x reference)
import functools

import jax
import jax.numpy as jnp
from jax import lax
from jax.experimental import pallas as pl
from jax.experimental.pallas import tpu as pltpu
from jax.experimental.pallas import tpu_sc as plsc

B, N, C = 4, 4096, 128
S = 1024
K = 32
NC, NS, L = 2, 16, 16
NW = NC * NS
RPW = (B * S) // NW
WPB = NW // B
NCH = N // L
GCH = 128


def _fps_body(x_ref, y_ref, z_ref, nx_ref, ny_ref, nz_ref):
    x = x_ref[...]
    y = y_ref[...]
    z = z_ref[...]
    npos = (lax.broadcasted_iota(jnp.int32, (32, 128), 0) * 128
            + lax.broadcasted_iota(jnp.int32, (32, 128), 1))
    slot = (lax.broadcasted_iota(jnp.int32, (8, 128), 0) * 128
            + lax.broadcasted_iota(jnp.int32, (8, 128), 1))
    nx_ref[...] = jnp.zeros((B, 8, 128), jnp.float32)
    ny_ref[...] = jnp.zeros((B, 8, 128), jnp.float32)
    nz_ref[...] = jnp.zeros((B, 8, 128), jnp.float32)

    def body(i, carry):
        dists, far = carry
        oh = npos[None] == far
        cx = jnp.sum(jnp.where(oh, x, 0.0), axis=(1, 2), keepdims=True)
        cy = jnp.sum(jnp.where(oh, y, 0.0), axis=(1, 2), keepdims=True)
        cz = jnp.sum(jnp.where(oh, z, 0.0), axis=(1, 2), keepdims=True)
        dx = x - cx
        dy = y - cy
        dz = z - cz
        d = dx * dx + dy * dy + dz * dz
        dists = jnp.minimum(dists, d)
        m = jnp.max(dists, axis=(1, 2), keepdims=True)
        farn = jnp.min(jnp.where(dists == m, npos[None], N),
                       axis=(1, 2), keepdims=True)
        sl = (slot == i)[None]
        nx_ref[...] += jnp.where(sl, cx, 0.0)
        ny_ref[...] += jnp.where(sl, cy, 0.0)
        nz_ref[...] += jnp.where(sl, cz, 0.0)
        return dists, farn

    lax.fori_loop(0, S, body,
                  (jnp.full((B, 32, 128), 1e10, jnp.float32),
                   jnp.zeros((B, 1, 1), jnp.int32)))


def _fps(x, y, z):
    return pl.pallas_call(
        _fps_body,
        out_shape=[jax.ShapeDtypeStruct((B, 8, 128), jnp.float32)] * 3,
    )(x, y, z)


_MESH = plsc.VectorSubcoreMesh(core_axis_name="c", subcore_axis_name="s",
                               num_cores=NC, num_subcores=NS)


def _bf16r(x):
    u = lax.bitcast_convert_type(x, jnp.int32)
    r = (u + 32767 + ((u >> 16) & 1)) & jnp.int32(-65536)
    return lax.bitcast_convert_type(r, jnp.float32)


def _ball_body(xyzt_hbm, newt_hbm, r2_hbm, idx_hbm, gx_hbm,
               xyz_v, new_v, r2_v, idx_v, gx_v, buf_v, xx_v, bxyz_v):
    c = lax.axis_index("c")
    s = lax.axis_index("s")
    w = s * NC + c
    b = w // WPB

    pltpu.sync_copy(xyzt_hbm.at[b], xyz_v)
    pltpu.sync_copy(newt_hbm.at[b], new_v)
    pltpu.sync_copy(r2_hbm, r2_v)
    r2 = r2_v[...]
    iota = lax.iota(jnp.int32, L)

    def pre_fn(ch, carry):
        xc = xyz_v[pl.ds(ch * L, L)]
        yc = xyz_v[pl.ds(N + ch * L, L)]
        zc = xyz_v[pl.ds(2 * N + ch * L, L)]
        xx_v[pl.ds(ch * L, L)] = xc * xc + yc * yc + zc * zc
        bxyz_v[pl.ds(ch * L, L)] = _bf16r(xc)
        bxyz_v[pl.ds(N + ch * L, L)] = _bf16r(yc)
        bxyz_v[pl.ds(2 * N + ch * L, L)] = _bf16r(zc)
        return carry

    lax.fori_loop(0, NCH, pre_fn, jnp.int32(0))

    r8 = w % WPB

    def row_fn(j, carry):
        jj = jnp.full((L,), j * WPB + r8, jnp.int32)
        cx = plsc.load_gather(new_v, [jj])
        cy = plsc.load_gather(new_v, [jj + S])
        cz = plsc.load_gather(new_v, [jj + 2 * S])
        ss = cx * cx + cy * cy + cz * cz
        bcx = _bf16r(cx)
        bcy = _bf16r(cy)
        bcz = _bf16r(cz)

        def cond(st):
            cnt, ch = st
            return jnp.logical_and(cnt < K, ch < NCH)

        def step(st):
            cnt, ch = st
            xx = xx_v[pl.ds(ch * L, L)]
            bx = bxyz_v[pl.ds(ch * L, L)]
            by = bxyz_v[pl.ds(N + ch * L, L)]
            bz = bxyz_v[pl.ds(2 * N + ch * L, L)]
            dot = bcx * bx + bcy * by + bcz * bz
            d2 = (ss + xx) - 2.0 * dot
            msk = d2 < r2
            pc = plsc.all_reduce_population_count(msk)[0]

            @pl.when(pc > 0)
            def _():
                cs = plsc.cumsum(msk.astype(jnp.int32))
                pos = jnp.maximum(cnt + cs - 1, 0)
                plsc.store_scatter(buf_v, [pos], ch * L + iota, mask=msk)

            return cnt + pc, ch + 1

        cnt, _ = lax.while_loop(cond, step, (jnp.int32(0), jnp.int32(0)))
        cntv = jnp.full((L,), cnt, jnp.int32)
        v0 = buf_v[pl.ds(0, L)]
        v1 = buf_v[pl.ds(L, L)]
        first = jnp.full((L,), v0[0], jnp.int32)
        first = jnp.where(cntv > 0, first, 0)
        i0 = jnp.clip(jnp.where(iota < cntv, v0, first), 0, N - 1)
        i1 = jnp.clip(jnp.where(iota + L < cntv, v1, first), 0, N - 1)
        idx_v[pl.ds(j * K, L)] = i0
        idx_v[pl.ds(j * K + L, L)] = i1
        for h, iv in ((0, i0), (1, i1)):
            gxx = plsc.load_gather(xyz_v, [iv]) - cx
            gxy = plsc.load_gather(xyz_v, [iv + N]) - cy
            gxz = plsc.load_gather(xyz_v, [iv + 2 * N]) - cz
            gx_v[pl.ds(j * 3 * K + h * L, L)] = gxx
            gx_v[pl.ds(j * 3 * K + K + h * L, L)] = gxy
            gx_v[pl.ds(j * 3 * K + 2 * K + h * L, L)] = gxz
        return carry

    lax.fori_loop(0, RPW, row_fn, jnp.int32(0))
    pltpu.sync_copy(idx_v, idx_hbm.at[pl.ds(w * RPW * K, RPW * K)])
    pltpu.sync_copy(gx_v, gx_hbm.at[pl.ds(w * RPW * 3 * K, RPW * 3 * K)])


@functools.partial(
    pl.kernel,
    out_type=(jax.ShapeDtypeStruct((B * S * K,), jnp.int32),
              jax.ShapeDtypeStruct((B * S * 3 * K,), jnp.float32)),
    mesh=_MESH,
    compiler_params=pltpu.CompilerParams(needs_layout_passes=False),
    scratch_types=[
        pltpu.VMEM((3 * N,), jnp.float32),
        pltpu.VMEM((3 * S,), jnp.float32),
        pltpu.VMEM((L,), jnp.float32),
        pltpu.VMEM((RPW * K,), jnp.int32),
        pltpu.VMEM((RPW * 3 * K,), jnp.float32),
        pltpu.VMEM((48,), jnp.int32),
        pltpu.VMEM((N,), jnp.float32),
        pltpu.VMEM((3 * N,), jnp.float32),
    ],
)
def _ball(*args):
    _ball_body(*args)


def _gather_body(pts_hbm, idx_hbm, out_hbm, idxr_v, idxg_v, rows_v, sem):
    c = lax.axis_index("c")
    s = lax.axis_index("s")
    w = s * NC + c
    per_w = (B * S * K) // NW
    b = w // WPB
    off = jnp.full((L,), b * N, jnp.int32)

    def chunk_fn(ch, carry):
        base = w * per_w + ch * GCH
        pltpu.sync_copy(idx_hbm.at[pl.ds(base, GCH)], idxr_v)
        for t in range(GCH // L):
            idxg_v[pl.ds(t * L, L)] = (
                jnp.clip(idxr_v[pl.ds(t * L, L)], 0, N - 1) + off)
        pltpu.async_copy(pts_hbm.at[idxg_v], rows_v, sem).wait()
        pltpu.sync_copy(rows_v, out_hbm.at[pl.ds(base, GCH)])
        return carry

    lax.fori_loop(0, per_w // GCH, chunk_fn, jnp.int32(0))


@functools.partial(
    pl.kernel,
    out_type=jax.ShapeDtypeStruct((B * S * K, C), jnp.float32),
    mesh=_MESH,
    compiler_params=pltpu.CompilerParams(needs_layout_passes=False),
    scratch_types=[
        pltpu.VMEM((GCH,), jnp.int32),
        pltpu.VMEM((GCH,), jnp.int32),
        pltpu.VMEM((GCH, C), jnp.float32),
        pltpu.SemaphoreType.DMA,
    ],
)
def _gather_pts(*args):
    _gather_body(*args)


def kernel(npoint, radius, xyz, points):
    del npoint
    xt = jnp.transpose(xyz, (0, 2, 1))
    xr = xt.reshape(B, 3, 32, 128)
    nx, ny, nz = _fps(xr[:, 0], xr[:, 1], xr[:, 2])
    nx = nx.reshape(B, S)
    ny = ny.reshape(B, S)
    nz = nz.reshape(B, S)
    new_xyz = jnp.stack([nx, ny, nz], axis=-1)
    newt = jnp.stack([nx, ny, nz], axis=1).reshape(B, 3 * S)
    r2 = jnp.full((L,), radius * radius, jnp.float32)
    idx_flat, gx_flat = _ball(xt.reshape(B, 3 * N), newt, r2)
    idx = (idx_flat.reshape(B, WPB, RPW, K)
           .transpose(0, 2, 1, 3).reshape(B, S, K))
    gx = (gx_flat.reshape(B, WPB, RPW, 3, K)
          .transpose(0, 2, 1, 3, 4).reshape(B, S, 3, K))
    grouped_xyz = jnp.transpose(gx, (0, 1, 3, 2))
    new_points = _gather_pts(points.reshape(B * N, C),
                             idx.reshape(B * S * K)).reshape(B, S, K, C)
    return new_xyz, new_points, idx, grouped_xyz


# speedup vs baseline: 1.0429x; 1.0429x over previous
import functools

import jax
import jax.numpy as jnp
from jax import lax
from jax.experimental import pallas as pl
from jax.experimental.pallas import tpu as pltpu
from jax.experimental.pallas import tpu_sc as plsc

B, N, C = 4, 4096, 128
S = 1024
K = 32
NC, NS, L = 2, 16, 16
NW = NC * NS
RPW = (B * S) // NW
WPB = NW // B
NCH = N // L
GCH = 128


def _fps_body(x_ref, y_ref, z_ref, nx_ref, ny_ref, nz_ref):
    x = x_ref[...]
    y = y_ref[...]
    z = z_ref[...]
    npos = (lax.broadcasted_iota(jnp.int32, (32, 128), 0) * 128
            + lax.broadcasted_iota(jnp.int32, (32, 128), 1))
    slot = (lax.broadcasted_iota(jnp.int32, (8, 128), 0) * 128
            + lax.broadcasted_iota(jnp.int32, (8, 128), 1))
    nx_ref[...] = jnp.zeros((B, 8, 128), jnp.float32)
    ny_ref[...] = jnp.zeros((B, 8, 128), jnp.float32)
    nz_ref[...] = jnp.zeros((B, 8, 128), jnp.float32)

    def body(i, carry):
        dists, far = carry
        oh = npos[None] == far
        cx = jnp.sum(jnp.where(oh, x, 0.0), axis=(1, 2), keepdims=True)
        cy = jnp.sum(jnp.where(oh, y, 0.0), axis=(1, 2), keepdims=True)
        cz = jnp.sum(jnp.where(oh, z, 0.0), axis=(1, 2), keepdims=True)
        dx = x - cx
        dy = y - cy
        dz = z - cz
        d = dx * dx + dy * dy + dz * dz
        dists = jnp.minimum(dists, d)
        m = jnp.max(dists, axis=(1, 2), keepdims=True)
        farn = jnp.min(jnp.where(dists == m, npos[None], N),
                       axis=(1, 2), keepdims=True)
        sl = (slot == i)[None]
        nx_ref[...] += jnp.where(sl, cx, 0.0)
        ny_ref[...] += jnp.where(sl, cy, 0.0)
        nz_ref[...] += jnp.where(sl, cz, 0.0)
        return dists, farn

    lax.fori_loop(0, S, body,
                  (jnp.full((B, 32, 128), 1e10, jnp.float32),
                   jnp.zeros((B, 1, 1), jnp.int32)))


def _fps(x, y, z):
    return pl.pallas_call(
        _fps_body,
        out_shape=[jax.ShapeDtypeStruct((B, 8, 128), jnp.float32)] * 3,
    )(x, y, z)


_MESH = plsc.VectorSubcoreMesh(core_axis_name="c", subcore_axis_name="s",
                               num_cores=NC, num_subcores=NS)


def _bf16r(x):
    u = lax.bitcast_convert_type(x, jnp.int32)
    r = (u + 32767 + ((u >> 16) & 1)) & jnp.int32(-65536)
    return lax.bitcast_convert_type(r, jnp.float32)


def _ball_body(xyzt_hbm, newt_hbm, r2_hbm, idx_hbm, gx_hbm,
               xyz_v, new_v, r2_v, idx_v, gx_v, buf_v, xx_v, bxyz_v):
    c = lax.axis_index("c")
    s = lax.axis_index("s")
    w = s * NC + c
    b = w // WPB

    pltpu.sync_copy(xyzt_hbm.at[b], xyz_v)
    pltpu.sync_copy(newt_hbm.at[b], new_v)
    pltpu.sync_copy(r2_hbm, r2_v)
    r2 = r2_v[...]
    iota = lax.iota(jnp.int32, L)

    def pre_fn(ch, carry):
        xc = xyz_v[pl.ds(ch * L, L)]
        yc = xyz_v[pl.ds(N + ch * L, L)]
        zc = xyz_v[pl.ds(2 * N + ch * L, L)]
        xx_v[pl.ds(ch * L, L)] = xc * xc + yc * yc + zc * zc
        bxyz_v[pl.ds(ch * L, L)] = _bf16r(xc)
        bxyz_v[pl.ds(N + ch * L, L)] = _bf16r(yc)
        bxyz_v[pl.ds(2 * N + ch * L, L)] = _bf16r(zc)
        return carry

    lax.fori_loop(0, NCH, pre_fn, jnp.int32(0))

    r8 = w % WPB

    def row_fn(j, carry):
        jj = jnp.full((L,), j * WPB + r8, jnp.int32)
        cx = plsc.load_gather(new_v, [jj])
        cy = plsc.load_gather(new_v, [jj + S])
        cz = plsc.load_gather(new_v, [jj + 2 * S])
        ss = cx * cx + cy * cy + cz * cz
        bcx = _bf16r(cx)
        bcy = _bf16r(cy)
        bcz = _bf16r(cz)

        def cond(st):
            cnt, ch = st
            return jnp.logical_and(cnt < K, ch < NCH)

        def step(st):
            cnt, ch = st
            xx = xx_v[pl.ds(ch * L, L)]
            bx = bxyz_v[pl.ds(ch * L, L)]
            by = bxyz_v[pl.ds(N + ch * L, L)]
            bz = bxyz_v[pl.ds(2 * N + ch * L, L)]
            dot = bcx * bx + bcy * by + bcz * bz
            d2 = (ss + xx) - 2.0 * dot
            msk = d2 < r2
            cs = plsc.cumsum(msk.astype(jnp.int32))
            pos = jnp.maximum(cnt + cs - 1, 0)
            plsc.store_scatter(buf_v, [pos], ch * L + iota, mask=msk)
            cnt = cnt + cs[L - 1]
            return cnt, ch + 1

        cnt, _ = lax.while_loop(cond, step, (jnp.int32(0), jnp.int32(0)))
        cntv = jnp.full((L,), cnt, jnp.int32)
        v0 = buf_v[pl.ds(0, L)]
        v1 = buf_v[pl.ds(L, L)]
        first = jnp.full((L,), v0[0], jnp.int32)
        first = jnp.where(cntv > 0, first, 0)
        i0 = jnp.clip(jnp.where(iota < cntv, v0, first), 0, N - 1)
        i1 = jnp.clip(jnp.where(iota + L < cntv, v1, first), 0, N - 1)
        idx_v[pl.ds(j * K, L)] = i0
        idx_v[pl.ds(j * K + L, L)] = i1
        for h, iv in ((0, i0), (1, i1)):
            gxx = plsc.load_gather(xyz_v, [iv]) - cx
            gxy = plsc.load_gather(xyz_v, [iv + N]) - cy
            gxz = plsc.load_gather(xyz_v, [iv + 2 * N]) - cz
            gx_v[pl.ds(j * 3 * K + h * L, L)] = gxx
            gx_v[pl.ds(j * 3 * K + K + h * L, L)] = gxy
            gx_v[pl.ds(j * 3 * K + 2 * K + h * L, L)] = gxz
        return carry

    lax.fori_loop(0, RPW, row_fn, jnp.int32(0))
    pltpu.sync_copy(idx_v, idx_hbm.at[pl.ds(w * RPW * K, RPW * K)])
    pltpu.sync_copy(gx_v, gx_hbm.at[pl.ds(w * RPW * 3 * K, RPW * 3 * K)])


@functools.partial(
    pl.kernel,
    out_type=(jax.ShapeDtypeStruct((B * S * K,), jnp.int32),
              jax.ShapeDtypeStruct((B * S * 3 * K,), jnp.float32)),
    mesh=_MESH,
    compiler_params=pltpu.CompilerParams(needs_layout_passes=False),
    scratch_types=[
        pltpu.VMEM((3 * N,), jnp.float32),
        pltpu.VMEM((3 * S,), jnp.float32),
        pltpu.VMEM((L,), jnp.float32),
        pltpu.VMEM((RPW * K,), jnp.int32),
        pltpu.VMEM((RPW * 3 * K,), jnp.float32),
        pltpu.VMEM((48,), jnp.int32),
        pltpu.VMEM((N,), jnp.float32),
        pltpu.VMEM((3 * N,), jnp.float32),
    ],
)
def _ball(*args):
    _ball_body(*args)


def _gather_body(pts_hbm, idx_hbm, out_hbm, idxr_v, idxg_v, rows_v, sem,
                 sem_o):
    c = lax.axis_index("c")
    s = lax.axis_index("s")
    w = s * NC + c
    per_w = (B * S * K) // NW
    nch = per_w // GCH
    b = w // WPB
    off = jnp.full((L,), b * N, jnp.int32)

    pltpu.sync_copy(idx_hbm.at[pl.ds(w * per_w, per_w)], idxr_v)
    for t in range(per_w // L):
        idxg_v[pl.ds(t * L, L)] = (
            jnp.clip(idxr_v[pl.ds(t * L, L)], 0, N - 1) + off)

    def chunk_fn(ch, carry):
        p = lax.rem(ch, 2)
        base = w * per_w + ch * GCH
        pltpu.async_copy(pts_hbm.at[idxg_v.at[pl.ds(ch * GCH, GCH)]],
                         rows_v.at[p], sem).wait()

        @pl.when(ch > 0)
        def _():
            pltpu.make_async_copy(
                rows_v.at[1 - p],
                out_hbm.at[pl.ds(base - GCH, GCH)], sem_o).wait()

        pltpu.async_copy(rows_v.at[p], out_hbm.at[pl.ds(base, GCH)], sem_o)
        return carry

    lax.fori_loop(0, nch, chunk_fn, jnp.int32(0))
    last = w * per_w + (nch - 1) * GCH
    pltpu.make_async_copy(rows_v.at[lax.rem(nch - 1, 2)],
                          out_hbm.at[pl.ds(last, GCH)], sem_o).wait()


@functools.partial(
    pl.kernel,
    out_type=jax.ShapeDtypeStruct((B * S * K, C), jnp.float32),
    mesh=_MESH,
    compiler_params=pltpu.CompilerParams(needs_layout_passes=False),
    scratch_types=[
        pltpu.VMEM(((B * S * K) // NW,), jnp.int32),
        pltpu.VMEM(((B * S * K) // NW,), jnp.int32),
        pltpu.VMEM((2, GCH, C), jnp.float32),
        pltpu.SemaphoreType.DMA,
        pltpu.SemaphoreType.DMA,
    ],
)
def _gather_pts(*args):
    _gather_body(*args)


def kernel(npoint, radius, xyz, points):
    del npoint
    xt = jnp.transpose(xyz, (0, 2, 1))
    xr = xt.reshape(B, 3, 32, 128)
    nx, ny, nz = _fps(xr[:, 0], xr[:, 1], xr[:, 2])
    nx = nx.reshape(B, S)
    ny = ny.reshape(B, S)
    nz = nz.reshape(B, S)
    new_xyz = jnp.stack([nx, ny, nz], axis=-1)
    newt = jnp.stack([nx, ny, nz], axis=1).reshape(B, 3 * S)
    r2 = jnp.full((L,), radius * radius, jnp.float32)
    idx_flat, gx_flat = _ball(xt.reshape(B, 3 * N), newt, r2)
    idx = (idx_flat.reshape(B, WPB, RPW, K)
           .transpose(0, 2, 1, 3).reshape(B, S, K))
    gx = (gx_flat.reshape(B, WPB, RPW, 3, K)
          .transpose(0, 2, 1, 3, 4).reshape(B, S, 3, K))
    grouped_xyz = jnp.transpose(gx, (0, 1, 3, 2))
    new_points = _gather_pts(points.reshape(B * N, C),
                             idx.reshape(B * S * K)).reshape(B, S, K, C)
    return new_xyz, new_points, idx, grouped_xyz


# speedup vs baseline: 1.2931x; 1.2399x over previous
import functools

import jax
import jax.numpy as jnp
from jax import lax
from jax.experimental import pallas as pl
from jax.experimental.pallas import tpu as pltpu
from jax.experimental.pallas import tpu_sc as plsc

B, N, C = 4, 4096, 128
S = 1024
K = 32
NC, NS, L = 2, 16, 16
NW = NC * NS
RPW = (B * S) // NW
WPB = NW // B
NCH = N // L
GCH = 128


def _fps_body(x_ref, y_ref, z_ref, nx_ref, ny_ref, nz_ref):
    x = x_ref[...]
    y = y_ref[...]
    z = z_ref[...]
    npos = (lax.broadcasted_iota(jnp.int32, (32, 128), 0) * 128
            + lax.broadcasted_iota(jnp.int32, (32, 128), 1))
    slot = (lax.broadcasted_iota(jnp.int32, (8, 128), 0) * 128
            + lax.broadcasted_iota(jnp.int32, (8, 128), 1))
    nx_ref[...] = jnp.zeros((B, 8, 128), jnp.float32)
    ny_ref[...] = jnp.zeros((B, 8, 128), jnp.float32)
    nz_ref[...] = jnp.zeros((B, 8, 128), jnp.float32)

    def body(i, carry):
        dists, far = carry
        oh = npos[None] == far
        cx = jnp.sum(jnp.where(oh, x, 0.0), axis=(1, 2), keepdims=True)
        cy = jnp.sum(jnp.where(oh, y, 0.0), axis=(1, 2), keepdims=True)
        cz = jnp.sum(jnp.where(oh, z, 0.0), axis=(1, 2), keepdims=True)
        dx = x - cx
        dy = y - cy
        dz = z - cz
        d = dx * dx + dy * dy + dz * dz
        dists = jnp.minimum(dists, d)
        m = jnp.max(dists, axis=(1, 2), keepdims=True)
        farn = jnp.min(jnp.where(dists == m, npos[None], N),
                       axis=(1, 2), keepdims=True)
        sl = (slot == i)[None]
        nx_ref[...] += jnp.where(sl, cx, 0.0)
        ny_ref[...] += jnp.where(sl, cy, 0.0)
        nz_ref[...] += jnp.where(sl, cz, 0.0)
        return dists, farn

    lax.fori_loop(0, S, body,
                  (jnp.full((B, 32, 128), 1e10, jnp.float32),
                   jnp.zeros((B, 1, 1), jnp.int32)))


def _fps(x, y, z):
    return pl.pallas_call(
        _fps_body,
        out_shape=[jax.ShapeDtypeStruct((B, 8, 128), jnp.float32)] * 3,
    )(x, y, z)


_MESH = plsc.VectorSubcoreMesh(core_axis_name="c", subcore_axis_name="s",
                               num_cores=NC, num_subcores=NS)


def _bf16r(x):
    u = lax.bitcast_convert_type(x, jnp.int32)
    r = (u + 32767 + ((u >> 16) & 1)) & jnp.int32(-65536)
    return lax.bitcast_convert_type(r, jnp.float32)


def _ball_body(xyzt_hbm, newt_hbm, r2_hbm, idx_hbm, gx_hbm,
               xyz_v, new_v, r2_v, idx_v, gx_v, buf_v, xx_v, bxyz_v):
    c = lax.axis_index("c")
    s = lax.axis_index("s")
    w = s * NC + c
    b = w // WPB

    pltpu.sync_copy(xyzt_hbm.at[b], xyz_v)
    pltpu.sync_copy(newt_hbm.at[b], new_v)
    pltpu.sync_copy(r2_hbm, r2_v)
    r2 = r2_v[...]
    iota = lax.iota(jnp.int32, L)

    def pre_fn(ch, carry):
        xc = xyz_v[pl.ds(ch * L, L)]
        yc = xyz_v[pl.ds(N + ch * L, L)]
        zc = xyz_v[pl.ds(2 * N + ch * L, L)]
        xx_v[pl.ds(ch * L, L)] = xc * xc + yc * yc + zc * zc
        bxyz_v[pl.ds(ch * L, L)] = _bf16r(xc)
        bxyz_v[pl.ds(N + ch * L, L)] = _bf16r(yc)
        bxyz_v[pl.ds(2 * N + ch * L, L)] = _bf16r(zc)
        return carry

    lax.fori_loop(0, NCH, pre_fn, jnp.int32(0))

    r8 = w % WPB

    def row_fn(j, carry):
        jj = jnp.full((L,), j * WPB + r8, jnp.int32)
        cx = plsc.load_gather(new_v, [jj])
        cy = plsc.load_gather(new_v, [jj + S])
        cz = plsc.load_gather(new_v, [jj + 2 * S])
        ss = cx * cx + cy * cy + cz * cz
        bcx = _bf16r(cx)
        bcy = _bf16r(cy)
        bcz = _bf16r(cz)

        def cond(st):
            cnt, ch = st
            return jnp.logical_and(cnt < K, ch < NCH)

        def step(st):
            cnt, ch = st
            xx0 = xx_v[pl.ds(ch * L, L)]
            bx0 = bxyz_v[pl.ds(ch * L, L)]
            by0 = bxyz_v[pl.ds(N + ch * L, L)]
            bz0 = bxyz_v[pl.ds(2 * N + ch * L, L)]
            xx1 = xx_v[pl.ds(ch * L + L, L)]
            bx1 = bxyz_v[pl.ds(ch * L + L, L)]
            by1 = bxyz_v[pl.ds(N + ch * L + L, L)]
            bz1 = bxyz_v[pl.ds(2 * N + ch * L + L, L)]
            d20 = (ss + xx0) - 2.0 * (bcx * bx0 + bcy * by0 + bcz * bz0)
            d21 = (ss + xx1) - 2.0 * (bcx * bx1 + bcy * by1 + bcz * bz1)
            msk0 = d20 < r2
            msk1 = d21 < r2
            cs0 = plsc.cumsum(msk0.astype(jnp.int32))
            cs1 = plsc.cumsum(msk1.astype(jnp.int32))
            pos0 = jnp.maximum(cnt + cs0 - 1, 0)
            plsc.store_scatter(buf_v, [pos0], ch * L + iota, mask=msk0)
            cnt = cnt + cs0[L - 1]
            pos1 = jnp.maximum(cnt + cs1 - 1, 0)
            plsc.store_scatter(buf_v, [pos1], ch * L + L + iota, mask=msk1)
            cnt = cnt + cs1[L - 1]
            return cnt, ch + 2

        cnt, _ = lax.while_loop(cond, step, (jnp.int32(0), jnp.int32(0)))
        cntv = jnp.full((L,), cnt, jnp.int32)
        v0 = buf_v[pl.ds(0, L)]
        v1 = buf_v[pl.ds(L, L)]
        first = jnp.full((L,), v0[0], jnp.int32)
        first = jnp.where(cntv > 0, first, 0)
        i0 = jnp.clip(jnp.where(iota < cntv, v0, first), 0, N - 1)
        i1 = jnp.clip(jnp.where(iota + L < cntv, v1, first), 0, N - 1)
        idx_v[pl.ds(j * K, L)] = i0
        idx_v[pl.ds(j * K + L, L)] = i1
        for h, iv in ((0, i0), (1, i1)):
            gxx = plsc.load_gather(xyz_v, [iv]) - cx
            gxy = plsc.load_gather(xyz_v, [iv + N]) - cy
            gxz = plsc.load_gather(xyz_v, [iv + 2 * N]) - cz
            gx_v[pl.ds(j * 3 * K + h * L, L)] = gxx
            gx_v[pl.ds(j * 3 * K + K + h * L, L)] = gxy
            gx_v[pl.ds(j * 3 * K + 2 * K + h * L, L)] = gxz
        return carry

    lax.fori_loop(0, RPW, row_fn, jnp.int32(0))
    pltpu.sync_copy(idx_v, idx_hbm.at[pl.ds(w * RPW * K, RPW * K)])
    pltpu.sync_copy(gx_v, gx_hbm.at[pl.ds(w * RPW * 3 * K, RPW * 3 * K)])


@functools.partial(
    pl.kernel,
    out_type=(jax.ShapeDtypeStruct((B * S * K,), jnp.int32),
              jax.ShapeDtypeStruct((B * S * 3 * K,), jnp.float32)),
    mesh=_MESH,
    compiler_params=pltpu.CompilerParams(needs_layout_passes=False),
    scratch_types=[
        pltpu.VMEM((3 * N,), jnp.float32),
        pltpu.VMEM((3 * S,), jnp.float32),
        pltpu.VMEM((L,), jnp.float32),
        pltpu.VMEM((RPW * K,), jnp.int32),
        pltpu.VMEM((RPW * 3 * K,), jnp.float32),
        pltpu.VMEM((80,), jnp.int32),
        pltpu.VMEM((N,), jnp.float32),
        pltpu.VMEM((3 * N,), jnp.float32),
    ],
)
def _ball(*args):
    _ball_body(*args)


def _gather_body(pts_hbm, idx_hbm, out_hbm, idxr_v, idxg_v, rows_v, sem,
                 sem_o):
    c = lax.axis_index("c")
    s = lax.axis_index("s")
    w = s * NC + c
    per_w = (B * S * K) // NW
    nch = per_w // GCH
    b = w // WPB
    off = jnp.full((L,), b * N, jnp.int32)

    pltpu.sync_copy(idx_hbm.at[pl.ds(w * per_w, per_w)], idxr_v)
    for t in range(per_w // L):
        idxg_v[pl.ds(t * L, L)] = (
            jnp.clip(idxr_v[pl.ds(t * L, L)], 0, N - 1) + off)

    def chunk_fn(ch, carry):
        p = lax.rem(ch, 2)
        base = w * per_w + ch * GCH
        pltpu.async_copy(pts_hbm.at[idxg_v.at[pl.ds(ch * GCH, GCH)]],
                         rows_v.at[p], sem).wait()

        @pl.when(ch > 0)
        def _():
            pltpu.make_async_copy(
                rows_v.at[1 - p],
                out_hbm.at[pl.ds(base - GCH, GCH)], sem_o).wait()

        pltpu.async_copy(rows_v.at[p], out_hbm.at[pl.ds(base, GCH)], sem_o)
        return carry

    lax.fori_loop(0, nch, chunk_fn, jnp.int32(0))
    last = w * per_w + (nch - 1) * GCH
    pltpu.make_async_copy(rows_v.at[lax.rem(nch - 1, 2)],
                          out_hbm.at[pl.ds(last, GCH)], sem_o).wait()


@functools.partial(
    pl.kernel,
    out_type=jax.ShapeDtypeStruct((B * S * K, C), jnp.float32),
    mesh=_MESH,
    compiler_params=pltpu.CompilerParams(needs_layout_passes=False),
    scratch_types=[
        pltpu.VMEM(((B * S * K) // NW,), jnp.int32),
        pltpu.VMEM(((B * S * K) // NW,), jnp.int32),
        pltpu.VMEM((2, GCH, C), jnp.float32),
        pltpu.SemaphoreType.DMA,
        pltpu.SemaphoreType.DMA,
    ],
)
def _gather_pts(*args):
    _gather_body(*args)


def kernel(npoint, radius, xyz, points):
    del npoint
    xt = jnp.transpose(xyz, (0, 2, 1))
    xr = xt.reshape(B, 3, 32, 128)
    nx, ny, nz = _fps(xr[:, 0], xr[:, 1], xr[:, 2])
    nx = nx.reshape(B, S)
    ny = ny.reshape(B, S)
    nz = nz.reshape(B, S)
    new_xyz = jnp.stack([nx, ny, nz], axis=-1)
    newt = jnp.stack([nx, ny, nz], axis=1).reshape(B, 3 * S)
    r2 = jnp.full((L,), radius * radius, jnp.float32)
    idx_flat, gx_flat = _ball(xt.reshape(B, 3 * N), newt, r2)
    idx = (idx_flat.reshape(B, WPB, RPW, K)
           .transpose(0, 2, 1, 3).reshape(B, S, K))
    gx = (gx_flat.reshape(B, WPB, RPW, 3, K)
          .transpose(0, 2, 1, 3, 4).reshape(B, S, 3, K))
    grouped_xyz = jnp.transpose(gx, (0, 1, 3, 2))
    new_points = _gather_pts(points.reshape(B * N, C),
                             idx.reshape(B * S * K)).reshape(B, S, K, C)
    return new_xyz, new_points, idx, grouped_xyz


# speedup vs baseline: 1.5029x; 1.1623x over previous
import functools

import jax
import jax.numpy as jnp
from jax import lax
from jax.experimental import pallas as pl
from jax.experimental.pallas import tpu as pltpu
from jax.experimental.pallas import tpu_sc as plsc

B, N, C = 4, 4096, 128
S = 1024
K = 32
NC, NS, L = 2, 16, 16
NW = NC * NS
RPW = (B * S) // NW
WPB = NW // B
NCH = N // L
GCH = 128
UNROLL = 4


def _fps_body(x_ref, y_ref, z_ref, nx_ref, ny_ref, nz_ref):
    x = x_ref[...]
    y = y_ref[...]
    z = z_ref[...]
    npos = (lax.broadcasted_iota(jnp.int32, (32, 128), 0) * 128
            + lax.broadcasted_iota(jnp.int32, (32, 128), 1))
    slot = (lax.broadcasted_iota(jnp.int32, (8, 128), 0) * 128
            + lax.broadcasted_iota(jnp.int32, (8, 128), 1))
    nx_ref[...] = jnp.zeros((B, 8, 128), jnp.float32)
    ny_ref[...] = jnp.zeros((B, 8, 128), jnp.float32)
    nz_ref[...] = jnp.zeros((B, 8, 128), jnp.float32)

    def body(i, carry):
        dists, far = carry
        oh = npos[None] == far
        cx = jnp.sum(jnp.where(oh, x, 0.0), axis=(1, 2), keepdims=True)
        cy = jnp.sum(jnp.where(oh, y, 0.0), axis=(1, 2), keepdims=True)
        cz = jnp.sum(jnp.where(oh, z, 0.0), axis=(1, 2), keepdims=True)
        dx = x - cx
        dy = y - cy
        dz = z - cz
        d = dx * dx + dy * dy + dz * dz
        dists = jnp.minimum(dists, d)
        m = jnp.max(dists, axis=(1, 2), keepdims=True)
        farn = jnp.min(jnp.where(dists == m, npos[None], N),
                       axis=(1, 2), keepdims=True)
        sl = (slot == i)[None]
        nx_ref[...] += jnp.where(sl, cx, 0.0)
        ny_ref[...] += jnp.where(sl, cy, 0.0)
        nz_ref[...] += jnp.where(sl, cz, 0.0)
        return dists, farn

    lax.fori_loop(0, S, body,
                  (jnp.full((B, 32, 128), 1e10, jnp.float32),
                   jnp.zeros((B, 1, 1), jnp.int32)))


def _fps(x, y, z):
    return pl.pallas_call(
        _fps_body,
        out_shape=[jax.ShapeDtypeStruct((B, 8, 128), jnp.float32)] * 3,
    )(x, y, z)


_MESH = plsc.VectorSubcoreMesh(core_axis_name="c", subcore_axis_name="s",
                               num_cores=NC, num_subcores=NS)


def _bf16r(x):
    u = lax.bitcast_convert_type(x, jnp.int32)
    r = (u + 32767 + ((u >> 16) & 1)) & jnp.int32(-65536)
    return lax.bitcast_convert_type(r, jnp.float32)


def _ball_body(xyzt_hbm, newt_hbm, r2_hbm, idx_hbm, gx_hbm,
               xyz_v, new_v, r2_v, idx_v, gx_v, buf_v, xx_v, bxyz_v):
    c = lax.axis_index("c")
    s = lax.axis_index("s")
    w = s * NC + c
    b = w // WPB

    pltpu.sync_copy(xyzt_hbm.at[b], xyz_v)
    pltpu.sync_copy(newt_hbm.at[b], new_v)
    pltpu.sync_copy(r2_hbm, r2_v)
    r2 = r2_v[...]
    iota = lax.iota(jnp.int32, L)

    def pre_fn(ch, carry):
        xc = xyz_v[pl.ds(ch * L, L)]
        yc = xyz_v[pl.ds(N + ch * L, L)]
        zc = xyz_v[pl.ds(2 * N + ch * L, L)]
        xx_v[pl.ds(ch * L, L)] = xc * xc + yc * yc + zc * zc
        bxyz_v[pl.ds(ch * L, L)] = _bf16r(xc)
        bxyz_v[pl.ds(N + ch * L, L)] = _bf16r(yc)
        bxyz_v[pl.ds(2 * N + ch * L, L)] = _bf16r(zc)
        return carry

    lax.fori_loop(0, NCH, pre_fn, jnp.int32(0))

    r8 = w % WPB

    def row_fn(j, carry):
        jj = jnp.full((L,), j * WPB + r8, jnp.int32)
        cx = plsc.load_gather(new_v, [jj])
        cy = plsc.load_gather(new_v, [jj + S])
        cz = plsc.load_gather(new_v, [jj + 2 * S])
        ss = cx * cx + cy * cy + cz * cz
        bcx = _bf16r(cx)
        bcy = _bf16r(cy)
        bcz = _bf16r(cz)

        def cond(st):
            cnt, ch = st
            return jnp.logical_and(cnt < K, ch < NCH)

        def step(st):
            cnt, ch = st
            msks, css = [], []
            for u in range(UNROLL):
                o = ch * L + u * L
                xx = xx_v[pl.ds(o, L)]
                bx = bxyz_v[pl.ds(o, L)]
                by = bxyz_v[pl.ds(N + o, L)]
                bz = bxyz_v[pl.ds(2 * N + o, L)]
                d2 = (ss + xx) - 2.0 * (bcx * bx + bcy * by + bcz * bz)
                msks.append(d2 < r2)
                css.append(plsc.cumsum(msks[-1].astype(jnp.int32)))
            for u in range(UNROLL):
                pos = jnp.maximum(cnt + css[u] - 1, 0)
                plsc.store_scatter(buf_v, [pos], ch * L + u * L + iota,
                                   mask=msks[u])
                cnt = cnt + css[u][L - 1]
            return cnt, ch + UNROLL

        cnt, _ = lax.while_loop(cond, step, (jnp.int32(0), jnp.int32(0)))
        cntv = jnp.full((L,), cnt, jnp.int32)
        v0 = buf_v[pl.ds(0, L)]
        v1 = buf_v[pl.ds(L, L)]
        first = jnp.full((L,), v0[0], jnp.int32)
        first = jnp.where(cntv > 0, first, 0)
        i0 = jnp.clip(jnp.where(iota < cntv, v0, first), 0, N - 1)
        i1 = jnp.clip(jnp.where(iota + L < cntv, v1, first), 0, N - 1)
        idx_v[pl.ds(j * K, L)] = i0
        idx_v[pl.ds(j * K + L, L)] = i1
        for h, iv in ((0, i0), (1, i1)):
            gxx = plsc.load_gather(xyz_v, [iv]) - cx
            gxy = plsc.load_gather(xyz_v, [iv + N]) - cy
            gxz = plsc.load_gather(xyz_v, [iv + 2 * N]) - cz
            gx_v[pl.ds(j * 3 * K + h * L, L)] = gxx
            gx_v[pl.ds(j * 3 * K + K + h * L, L)] = gxy
            gx_v[pl.ds(j * 3 * K + 2 * K + h * L, L)] = gxz
        return carry

    lax.fori_loop(0, RPW, row_fn, jnp.int32(0))
    pltpu.sync_copy(idx_v, idx_hbm.at[pl.ds(w * RPW * K, RPW * K)])
    pltpu.sync_copy(gx_v, gx_hbm.at[pl.ds(w * RPW * 3 * K, RPW * 3 * K)])


@functools.partial(
    pl.kernel,
    out_type=(jax.ShapeDtypeStruct((B * S * K,), jnp.int32),
              jax.ShapeDtypeStruct((B * S * 3 * K,), jnp.float32)),
    mesh=_MESH,
    compiler_params=pltpu.CompilerParams(needs_layout_passes=False),
    scratch_types=[
        pltpu.VMEM((3 * N,), jnp.float32),
        pltpu.VMEM((3 * S,), jnp.float32),
        pltpu.VMEM((L,), jnp.float32),
        pltpu.VMEM((RPW * K,), jnp.int32),
        pltpu.VMEM((RPW * 3 * K,), jnp.float32),
        pltpu.VMEM((K + UNROLL * L + L,), jnp.int32),
        pltpu.VMEM((N,), jnp.float32),
        pltpu.VMEM((3 * N,), jnp.float32),
    ],
)
def _ball(*args):
    _ball_body(*args)


def _gather_body(pts_hbm, idx_hbm, out_hbm, idxr_v, idxg_v, rows_v, sem,
                 sem_o):
    c = lax.axis_index("c")
    s = lax.axis_index("s")
    w = s * NC + c
    per_w = (B * S * K) // NW
    nch = per_w // GCH
    b = w // WPB
    off = jnp.full((L,), b * N, jnp.int32)

    pltpu.sync_copy(idx_hbm.at[pl.ds(w * per_w, per_w)], idxr_v)
    for t in range(per_w // L):
        idxg_v[pl.ds(t * L, L)] = (
            jnp.clip(idxr_v[pl.ds(t * L, L)], 0, N - 1) + off)

    def chunk_fn(ch, carry):
        p = lax.rem(ch, 2)
        base = w * per_w + ch * GCH
        pltpu.async_copy(pts_hbm.at[idxg_v.at[pl.ds(ch * GCH, GCH)]],
                         rows_v.at[p], sem).wait()

        @pl.when(ch > 0)
        def _():
            pltpu.make_async_copy(
                rows_v.at[1 - p],
                out_hbm.at[pl.ds(base - GCH, GCH)], sem_o).wait()

        pltpu.async_copy(rows_v.at[p], out_hbm.at[pl.ds(base, GCH)], sem_o)
        return carry

    lax.fori_loop(0, nch, chunk_fn, jnp.int32(0))
    last = w * per_w + (nch - 1) * GCH
    pltpu.make_async_copy(rows_v.at[lax.rem(nch - 1, 2)],
                          out_hbm.at[pl.ds(last, GCH)], sem_o).wait()


@functools.partial(
    pl.kernel,
    out_type=jax.ShapeDtypeStruct((B * S * K, C), jnp.float32),
    mesh=_MESH,
    compiler_params=pltpu.CompilerParams(needs_layout_passes=False),
    scratch_types=[
        pltpu.VMEM(((B * S * K) // NW,), jnp.int32),
        pltpu.VMEM(((B * S * K) // NW,), jnp.int32),
        pltpu.VMEM((2, GCH, C), jnp.float32),
        pltpu.SemaphoreType.DMA,
        pltpu.SemaphoreType.DMA,
    ],
)
def _gather_pts(*args):
    _gather_body(*args)


def kernel(npoint, radius, xyz, points):
    del npoint
    xt = jnp.transpose(xyz, (0, 2, 1))
    xr = xt.reshape(B, 3, 32, 128)
    nx, ny, nz = _fps(xr[:, 0], xr[:, 1], xr[:, 2])
    nx = nx.reshape(B, S)
    ny = ny.reshape(B, S)
    nz = nz.reshape(B, S)
    new_xyz = jnp.stack([nx, ny, nz], axis=-1)
    newt = jnp.stack([nx, ny, nz], axis=1).reshape(B, 3 * S)
    r2 = jnp.full((L,), radius * radius, jnp.float32)
    idx_flat, gx_flat = _ball(xt.reshape(B, 3 * N), newt, r2)
    idx = (idx_flat.reshape(B, WPB, RPW, K)
           .transpose(0, 2, 1, 3).reshape(B, S, K))
    gx = (gx_flat.reshape(B, WPB, RPW, 3, K)
          .transpose(0, 2, 1, 3, 4).reshape(B, S, 3, K))
    grouped_xyz = jnp.transpose(gx, (0, 1, 3, 2))
    new_points = _gather_pts(points.reshape(B * N, C),
                             idx.reshape(B * S * K)).reshape(B, S, K, C)
    return new_xyz, new_points, idx, grouped_xyz


# speedup vs baseline: 1.6151x; 1.0746x over previous
import functools

import jax
import jax.numpy as jnp
from jax import lax
from jax.experimental import pallas as pl
from jax.experimental.pallas import tpu as pltpu
from jax.experimental.pallas import tpu_sc as plsc

B, N, C = 4, 4096, 128
S = 1024
K = 32
NC, NS, L = 2, 16, 16
NW = NC * NS
RPW = (B * S) // NW
WPB = NW // B
NCH = N // L
GCH = 128
UNROLL = 8


def _fps_body(x_ref, y_ref, z_ref, nx_ref, ny_ref, nz_ref):
    x = x_ref[...]
    y = y_ref[...]
    z = z_ref[...]
    npos = (lax.broadcasted_iota(jnp.int32, (32, 128), 0) * 128
            + lax.broadcasted_iota(jnp.int32, (32, 128), 1))
    slot = (lax.broadcasted_iota(jnp.int32, (8, 128), 0) * 128
            + lax.broadcasted_iota(jnp.int32, (8, 128), 1))
    nx_ref[...] = jnp.zeros((B, 8, 128), jnp.float32)
    ny_ref[...] = jnp.zeros((B, 8, 128), jnp.float32)
    nz_ref[...] = jnp.zeros((B, 8, 128), jnp.float32)

    def body(i, carry):
        dists, far = carry
        oh = npos[None] == far
        cx = jnp.sum(jnp.where(oh, x, 0.0), axis=(1, 2), keepdims=True)
        cy = jnp.sum(jnp.where(oh, y, 0.0), axis=(1, 2), keepdims=True)
        cz = jnp.sum(jnp.where(oh, z, 0.0), axis=(1, 2), keepdims=True)
        dx = x - cx
        dy = y - cy
        dz = z - cz
        d = dx * dx + dy * dy + dz * dz
        dists = jnp.minimum(dists, d)
        m = jnp.max(dists, axis=(1, 2), keepdims=True)
        farn = jnp.min(jnp.where(dists == m, npos[None], N),
                       axis=(1, 2), keepdims=True)
        sl = (slot == i)[None]
        nx_ref[...] += jnp.where(sl, cx, 0.0)
        ny_ref[...] += jnp.where(sl, cy, 0.0)
        nz_ref[...] += jnp.where(sl, cz, 0.0)
        return dists, farn

    lax.fori_loop(0, S, body,
                  (jnp.full((B, 32, 128), 1e10, jnp.float32),
                   jnp.zeros((B, 1, 1), jnp.int32)))


def _fps(x, y, z):
    return pl.pallas_call(
        _fps_body,
        out_shape=[jax.ShapeDtypeStruct((B, 8, 128), jnp.float32)] * 3,
    )(x, y, z)


_MESH = plsc.VectorSubcoreMesh(core_axis_name="c", subcore_axis_name="s",
                               num_cores=NC, num_subcores=NS)


def _bf16r(x):
    u = lax.bitcast_convert_type(x, jnp.int32)
    r = (u + 32767 + ((u >> 16) & 1)) & jnp.int32(-65536)
    return lax.bitcast_convert_type(r, jnp.float32)


def _ball_body(xyzt_hbm, newt_hbm, r2_hbm, idx_hbm, gx_hbm,
               xyz_v, new_v, r2_v, idx_v, gx_v, buf_v, xx_v, bxyz_v):
    c = lax.axis_index("c")
    s = lax.axis_index("s")
    w = s * NC + c
    b = w // WPB

    pltpu.sync_copy(xyzt_hbm.at[b], xyz_v)
    pltpu.sync_copy(newt_hbm.at[b], new_v)
    pltpu.sync_copy(r2_hbm, r2_v)
    r2 = r2_v[...]
    iota = lax.iota(jnp.int32, L)

    def pre_fn(ch, carry):
        xc = xyz_v[pl.ds(ch * L, L)]
        yc = xyz_v[pl.ds(N + ch * L, L)]
        zc = xyz_v[pl.ds(2 * N + ch * L, L)]
        xx_v[pl.ds(ch * L, L)] = xc * xc + yc * yc + zc * zc
        bxyz_v[pl.ds(ch * L, L)] = _bf16r(xc)
        bxyz_v[pl.ds(N + ch * L, L)] = _bf16r(yc)
        bxyz_v[pl.ds(2 * N + ch * L, L)] = _bf16r(zc)
        return carry

    lax.fori_loop(0, NCH, pre_fn, jnp.int32(0))

    r8 = w % WPB

    def row_fn(j, carry):
        jj = jnp.full((L,), j * WPB + r8, jnp.int32)
        cx = plsc.load_gather(new_v, [jj])
        cy = plsc.load_gather(new_v, [jj + S])
        cz = plsc.load_gather(new_v, [jj + 2 * S])
        ss = cx * cx + cy * cy + cz * cz
        bcx = _bf16r(cx)
        bcy = _bf16r(cy)
        bcz = _bf16r(cz)

        def cond(st):
            cnt, ch = st
            return jnp.logical_and(cnt < K, ch < NCH)

        def step(st):
            cnt, ch = st
            msks, css = [], []
            for u in range(UNROLL):
                o = ch * L + u * L
                xx = xx_v[pl.ds(o, L)]
                bx = bxyz_v[pl.ds(o, L)]
                by = bxyz_v[pl.ds(N + o, L)]
                bz = bxyz_v[pl.ds(2 * N + o, L)]
                d2 = (ss + xx) - 2.0 * (bcx * bx + bcy * by + bcz * bz)
                msks.append(d2 < r2)
                css.append(plsc.cumsum(msks[-1].astype(jnp.int32)))
            for u in range(UNROLL):
                pos = jnp.maximum(cnt + css[u] - 1, 0)
                plsc.store_scatter(buf_v, [pos], ch * L + u * L + iota,
                                   mask=msks[u])
                cnt = cnt + css[u][L - 1]
            return cnt, ch + UNROLL

        cnt, _ = lax.while_loop(cond, step, (jnp.int32(0), jnp.int32(0)))
        cntv = jnp.full((L,), cnt, jnp.int32)
        v0 = buf_v[pl.ds(0, L)]
        v1 = buf_v[pl.ds(L, L)]
        first = jnp.full((L,), v0[0], jnp.int32)
        first = jnp.where(cntv > 0, first, 0)
        i0 = jnp.clip(jnp.where(iota < cntv, v0, first), 0, N - 1)
        i1 = jnp.clip(jnp.where(iota + L < cntv, v1, first), 0, N - 1)
        idx_v[pl.ds(j * K, L)] = i0
        idx_v[pl.ds(j * K + L, L)] = i1
        for h, iv in ((0, i0), (1, i1)):
            gxx = plsc.load_gather(xyz_v, [iv]) - cx
            gxy = plsc.load_gather(xyz_v, [iv + N]) - cy
            gxz = plsc.load_gather(xyz_v, [iv + 2 * N]) - cz
            gx_v[pl.ds(j * 3 * K + h * L, L)] = gxx
            gx_v[pl.ds(j * 3 * K + K + h * L, L)] = gxy
            gx_v[pl.ds(j * 3 * K + 2 * K + h * L, L)] = gxz
        return carry

    lax.fori_loop(0, RPW, row_fn, jnp.int32(0))
    pltpu.sync_copy(idx_v, idx_hbm.at[pl.ds(w * RPW * K, RPW * K)])
    pltpu.sync_copy(gx_v, gx_hbm.at[pl.ds(w * RPW * 3 * K, RPW * 3 * K)])


@functools.partial(
    pl.kernel,
    out_type=(jax.ShapeDtypeStruct((B * S * K,), jnp.int32),
              jax.ShapeDtypeStruct((B * S * 3 * K,), jnp.float32)),
    mesh=_MESH,
    compiler_params=pltpu.CompilerParams(needs_layout_passes=False),
    scratch_types=[
        pltpu.VMEM((3 * N,), jnp.float32),
        pltpu.VMEM((3 * S,), jnp.float32),
        pltpu.VMEM((L,), jnp.float32),
        pltpu.VMEM((RPW * K,), jnp.int32),
        pltpu.VMEM((RPW * 3 * K,), jnp.float32),
        pltpu.VMEM((K + UNROLL * L + L,), jnp.int32),
        pltpu.VMEM((N,), jnp.float32),
        pltpu.VMEM((3 * N,), jnp.float32),
    ],
)
def _ball(*args):
    _ball_body(*args)


def _gather_body(pts_hbm, idx_hbm, out_hbm, idxr_v, idxg_v, rows_v, sem,
                 sem_o):
    c = lax.axis_index("c")
    s = lax.axis_index("s")
    w = s * NC + c
    per_w = (B * S * K) // NW
    nch = per_w // GCH
    b = w // WPB
    off = jnp.full((L,), b * N, jnp.int32)

    pltpu.sync_copy(idx_hbm.at[pl.ds(w * per_w, per_w)], idxr_v)
    for t in range(per_w // L):
        idxg_v[pl.ds(t * L, L)] = (
            jnp.clip(idxr_v[pl.ds(t * L, L)], 0, N - 1) + off)

    def chunk_fn(ch, carry):
        p = lax.rem(ch, 2)
        base = w * per_w + ch * GCH
        pltpu.async_copy(pts_hbm.at[idxg_v.at[pl.ds(ch * GCH, GCH)]],
                         rows_v.at[p], sem).wait()

        @pl.when(ch > 0)
        def _():
            pltpu.make_async_copy(
                rows_v.at[1 - p],
                out_hbm.at[pl.ds(base - GCH, GCH)], sem_o).wait()

        pltpu.async_copy(rows_v.at[p], out_hbm.at[pl.ds(base, GCH)], sem_o)
        return carry

    lax.fori_loop(0, nch, chunk_fn, jnp.int32(0))
    last = w * per_w + (nch - 1) * GCH
    pltpu.make_async_copy(rows_v.at[lax.rem(nch - 1, 2)],
                          out_hbm.at[pl.ds(last, GCH)], sem_o).wait()


@functools.partial(
    pl.kernel,
    out_type=jax.ShapeDtypeStruct((B * S * K, C), jnp.float32),
    mesh=_MESH,
    compiler_params=pltpu.CompilerParams(needs_layout_passes=False),
    scratch_types=[
        pltpu.VMEM(((B * S * K) // NW,), jnp.int32),
        pltpu.VMEM(((B * S * K) // NW,), jnp.int32),
        pltpu.VMEM((2, GCH, C), jnp.float32),
        pltpu.SemaphoreType.DMA,
        pltpu.SemaphoreType.DMA,
    ],
)
def _gather_pts(*args):
    _gather_body(*args)


def kernel(npoint, radius, xyz, points):
    del npoint
    xt = jnp.transpose(xyz, (0, 2, 1))
    xr = xt.reshape(B, 3, 32, 128)
    nx, ny, nz = _fps(xr[:, 0], xr[:, 1], xr[:, 2])
    nx = nx.reshape(B, S)
    ny = ny.reshape(B, S)
    nz = nz.reshape(B, S)
    new_xyz = jnp.stack([nx, ny, nz], axis=-1)
    newt = jnp.stack([nx, ny, nz], axis=1).reshape(B, 3 * S)
    r2 = jnp.full((L,), radius * radius, jnp.float32)
    idx_flat, gx_flat = _ball(xt.reshape(B, 3 * N), newt, r2)
    idx = (idx_flat.reshape(B, WPB, RPW, K)
           .transpose(0, 2, 1, 3).reshape(B, S, K))
    gx = (gx_flat.reshape(B, WPB, RPW, 3, K)
          .transpose(0, 2, 1, 3, 4).reshape(B, S, 3, K))
    grouped_xyz = jnp.transpose(gx, (0, 1, 3, 2))
    new_points = _gather_pts(points.reshape(B * N, C),
                             idx.reshape(B * S * K)).reshape(B, S, K, C)
    return new_xyz, new_points, idx, grouped_xyz
